# Initial kernel scaffold; baseline (speedup 1.0000x reference)
import jax
import jax.numpy as jnp
from jax.experimental import pallas as pl


def kernel(x, edge_weight, params, edge_index):
    raise NotImplementedError("write your pallas kernel here")


# speedup vs baseline: 8.4480x; 8.4480x over previous
import functools

import jax
import jax.numpy as jnp
from jax import lax
from jax.experimental import pallas as pl
from jax.experimental.pallas import tpu as pltpu
from jax.experimental.pallas import tpu_sc as plsc

N = 10000
E = 320000
NW = 32
EPW = E // NW
K = 80
SB = 25
NSB = EPW // (SB * K)
MACC = 10240


@functools.cache
def _mesh():
    return plsc.VectorSubcoreMesh(core_axis_name="c", subcore_axis_name="s",
                                  num_cores=2, num_subcores=16)


_SC_PARAMS = pltpu.CompilerParams(needs_layout_passes=False)


def _deg_body(dst_hbm, w_hbm, out_hbm, dst_v, w_v, deg_v):
    c = lax.axis_index("c")
    s = lax.axis_index("s")
    wid = c * 16 + s
    pltpu.sync_copy(dst_hbm.at[wid], dst_v)
    pltpu.sync_copy(w_hbm.at[wid], w_v)

    zeros = jnp.zeros((16,), jnp.float32)

    def zero_body(i, carry):
        deg_v[pl.ds(i * 16, 16)] = zeros
        return carry

    lax.fori_loop(0, N // 16, zero_body, 0)

    def edge_body(i, carry):
        idx = dst_v[pl.ds(i * 16, 16)]
        wv = w_v[pl.ds(i * 16, 16)]
        plsc.addupdate_scatter(deg_v, [idx], wv)
        return carry

    lax.fori_loop(0, EPW // 16, edge_body, 0)
    pltpu.sync_copy(deg_v, out_hbm.at[wid])


@functools.cache
def _make_deg():
    return functools.partial(
        pl.kernel,
        mesh=_mesh(),
        out_type=jax.ShapeDtypeStruct((NW, N), jnp.float32),
        scratch_types=[
            pltpu.VMEM((EPW,), jnp.int32),
            pltpu.VMEM((EPW,), jnp.float32),
            pltpu.VMEM((N,), jnp.float32),
        ],
        compiler_params=_SC_PARAMS,
    )(_deg_body)


@functools.cache
def _make_agg(pack):
    assert pack in (1, 2, 4)
    shift = {1: 0, 2: 1, 4: 2}[pack]
    macc = MACC // pack
    rps = macc // 16

    scratch = [
        pltpu.VMEM((SB, K), jnp.int32),
        pltpu.VMEM((SB, K), jnp.int32),
        pltpu.VMEM((SB, K), jnp.float32),
    ]
    if pack > 1:
        scratch += [
            pltpu.VMEM((SB, K), jnp.int32),
            pltpu.VMEM((SB, K), jnp.int32),
        ]
    scratch += [
        pltpu.VMEM((K, 128), jnp.float32),
        pltpu.VMEM_SHARED((macc, 128), jnp.float32),
        pltpu.SemaphoreType.DMA,
    ]

    @functools.partial(
        pl.kernel,
        mesh=_mesh(),
        out_type=jax.ShapeDtypeStruct((2, macc, 128), jnp.float32),
        scratch_types=scratch,
        compiler_params=_SC_PARAMS,
    )
    def _agg(h_hbm, *args):
        if pack == 1:
            (src_hbm, dst_hbm, w_hbm, zero_hbm, out_hbm,
             src_b, dst_b, w_b, rows_v, acc_sh, sem) = args
            su_hbm = du_hbm = su_b = du_b = None
        else:
            (src_hbm, dst_hbm, su_hbm, du_hbm, w_hbm, zero_hbm, out_hbm,
             src_b, dst_b, w_b, su_b, du_b, rows_v, acc_sh, sem) = args
        c = lax.axis_index("c")
        s = lax.axis_index("s")
        wid = c * 16 + s
        row0 = pl.multiple_of(s * rps, 8)
        pltpu.sync_copy(zero_hbm.at[pl.ds(row0, rps)],
                        acc_sh.at[pl.ds(row0, rps)])
        plsc.subcore_barrier()

        zf = jnp.zeros((16,), jnp.float32)

        def chunk(jj, carry):
            pltpu.async_copy(h_hbm.at[src_b.at[jj]], rows_v, sem).wait()
            idx_j = jnp.full((16,), jj, dtype=jnp.int32)
            for e in range(K):
                idx_e = jnp.full((16,), e, dtype=jnp.int32)
                wsp = plsc.load_gather(w_b, [idx_j, idx_e])
                if pack == 1:
                    for k in range(8):
                        rows_v[e, pl.ds(k * 16, 16)] = (
                            rows_v[e, pl.ds(k * 16, 16)] * wsp)
                else:
                    g = [rows_v[e, pl.ds(k * 16, 16)] for k in range(8)]
                    su = plsc.load_gather(su_b, [idx_j, idx_e])
                    du = plsc.load_gather(du_b, [idx_j, idx_e])
                    if pack == 2:
                        sm = su == 1
                        m = [jnp.where(sm, g[k + 4], g[k]) * wsp
                             for k in range(4)]
                        dm = du == 1
                        for k in range(4):
                            rows_v[e, pl.ds(k * 16, 16)] = (
                                jnp.where(dm, zf, m[k]))
                            rows_v[e, pl.ds((k + 4) * 16, 16)] = (
                                jnp.where(dm, m[k], zf))
                    else:
                        b0 = jnp.bitwise_and(su, 1) == 1
                        b1 = jnp.bitwise_and(su, 2) == 2
                        m0 = jnp.where(b1, jnp.where(b0, g[6], g[4]),
                                       jnp.where(b0, g[2], g[0])) * wsp
                        m1 = jnp.where(b1, jnp.where(b0, g[7], g[5]),
                                       jnp.where(b0, g[3], g[1])) * wsp
                        for q in range(4):
                            dq = du == q
                            rows_v[e, pl.ds(2 * q * 16, 16)] = (
                                jnp.where(dq, m0, zf))
                            rows_v[e, pl.ds((2 * q + 1) * 16, 16)] = (
                                jnp.where(dq, m1, zf))
            pltpu.sync_copy(rows_v, acc_sh.at[dst_b.at[jj]], add=True)
            return carry

        def super_chunk(sb, carry):
            pltpu.sync_copy(src_hbm.at[wid, sb], src_b)
            pltpu.sync_copy(dst_hbm.at[wid, sb], dst_b)
            pltpu.sync_copy(w_hbm.at[wid, sb], w_b)
            if pack > 1:
                pltpu.sync_copy(su_hbm.at[wid, sb], su_b)
                pltpu.sync_copy(du_hbm.at[wid, sb], du_b)
            lax.fori_loop(0, SB, chunk, carry)
            return carry

        lax.fori_loop(0, NSB, super_chunk, 0)
        plsc.subcore_barrier()
        pltpu.sync_copy(acc_sh.at[pl.ds(row0, rps)],
                        out_hbm.at[c, pl.ds(row0, rps)])

    return _agg


def _bn(v, g, b, eps=1e-5):
    mu = jnp.mean(v, axis=0)
    var = jnp.var(v, axis=0)
    return (v - mu) / jnp.sqrt(var + eps) * g + b


def _tc_call(body, out_shapes, *args):
    return pl.pallas_call(body, out_shape=out_shapes)(*args)


def _stage1_body(x_ref, degp_ref, g_ref, b_ref, w_ref, hs_ref, dis_ref):
    deg = jnp.sum(degp_ref[...], axis=0) + 1.0
    dis = lax.rsqrt(deg)
    a = _bn(x_ref[...], g_ref[...], b_ref[...])
    h = jnp.dot(a, w_ref[...], preferred_element_type=jnp.float32)
    hs_ref[...] = h * dis[:, None]
    dis_ref[...] = dis[:, None]


def _stage_mid_body(p0_ref, p1_ref, hs_ref, dis_ref, bias_ref, g_ref, b_ref,
                    w_ref, out_ref):
    dis = dis_ref[...]
    agg = dis * (p0_ref[...] + p1_ref[...] + hs_ref[...]) + bias_ref[...]
    a = jax.nn.relu(_bn(agg, g_ref[...], b_ref[...]))
    h = jnp.dot(a, w_ref[...], preferred_element_type=jnp.float32)
    out_ref[...] = h * dis


def _stage_final_body(p0_ref, p1_ref, hs_ref, dis_ref, bias_ref, g5_ref,
                      b5_ref, l1w_ref, l1b_ref, g6_ref, b6_ref, l2w_ref,
                      l2b_ref, out_ref):
    dis = dis_ref[...]
    agg = dis * (p0_ref[...] + p1_ref[...] + hs_ref[...]) + bias_ref[...]
    h = _bn(agg, g5_ref[...], b5_ref[...])
    h = jnp.dot(jax.nn.relu(h), l1w_ref[...],
                preferred_element_type=jnp.float32) + l1b_ref[...]
    h = _bn(h, g6_ref[...], b6_ref[...])
    out_ref[...] = jnp.dot(jax.nn.relu(h), l2w_ref[...],
                           preferred_element_type=jnp.float32) + l2b_ref[...]


def _agg_call(hs, src4, dst4, w4, F):
    pack = 128 // F
    hp = hs.reshape(N // pack, 128)
    zeros = jnp.zeros((MACC // pack, 128), jnp.float32)
    if pack == 1:
        parts = _make_agg(1)(hp, src4, dst4, w4, zeros)
    else:
        shift = {2: 1, 4: 2}[pack]
        sp4 = src4 >> shift
        dp4 = dst4 >> shift
        su4 = src4 & (pack - 1)
        du4 = dst4 & (pack - 1)
        parts = _make_agg(pack)(hp, sp4, dp4, su4, du4, w4, zeros)
    p0 = parts[0, :N // pack].reshape(N, F)
    p1 = parts[1, :N // pack].reshape(N, F)
    return p0, p1


def kernel(x, edge_weight, params, edge_index):
    p = params
    src4 = edge_index[0].reshape(NW, NSB, SB, K)
    dst4 = edge_index[1].reshape(NW, NSB, SB, K)
    w4 = edge_weight.reshape(NW, NSB, SB, K)
    dst2 = edge_index[1].reshape(NW, EPW)
    w2 = edge_weight.reshape(NW, EPW)

    deg_parts = _make_deg()(dst2, w2)

    hs1, dis = _tc_call(
        _stage1_body,
        (jax.ShapeDtypeStruct((N, 128), jnp.float32),
         jax.ShapeDtypeStruct((N, 1), jnp.float32)),
        x, deg_parts, p['bn1_g'], p['bn1_b'], p['W1'])

    def conv_step(hs, F, bias, g, b, w_next, out_f):
        p0, p1 = _agg_call(hs, src4, dst4, w4, F)
        return _tc_call(
            _stage_mid_body,
            jax.ShapeDtypeStruct((N, out_f), jnp.float32),
            p0, p1, hs, dis, bias, g, b, w_next)

    hs2 = conv_step(hs1, 128, p['b1'], p['bn2_g'], p['bn2_b'], p['W2'], 128)
    hs3 = conv_step(hs2, 128, p['b2'], p['bn3_g'], p['bn3_b'], p['W3'], 64)
    hs4 = conv_step(hs3, 64, p['b3'], p['bn4_g'], p['bn4_b'], p['W4'], 32)

    p0, p1 = _agg_call(hs4, src4, dst4, w4, 32)
    out = _tc_call(
        _stage_final_body,
        jax.ShapeDtypeStruct((N, 40), jnp.float32),
        p0, p1, hs4, dis, p['b4'], p['bn5_g'], p['bn5_b'],
        p['lin1_W'], p['lin1_b'], p['bn6_g'], p['bn6_b'],
        p['lin2_W'], p['lin2_b'])
    return out


# speedup vs baseline: 16.3191x; 1.9317x over previous
import functools

import jax
import jax.numpy as jnp
from jax import lax
from jax.experimental import pallas as pl
from jax.experimental.pallas import tpu as pltpu
from jax.experimental.pallas import tpu_sc as plsc

N = 10000
E = 320000
NW = 32
EPW = E // NW
K = 80
SB = 25
NSB = EPW // (SB * K)
EG = 16


@functools.cache
def _mesh():
    return plsc.VectorSubcoreMesh(core_axis_name="c", subcore_axis_name="s",
                                  num_cores=2, num_subcores=16)


_SC_PARAMS = pltpu.CompilerParams(needs_layout_passes=False)


def _deg_body(dst_hbm, w_hbm, out_hbm, dst_v, w_v, deg_v):
    c = lax.axis_index("c")
    s = lax.axis_index("s")
    wid = c * 16 + s
    pltpu.sync_copy(dst_hbm.at[wid], dst_v)
    pltpu.sync_copy(w_hbm.at[wid], w_v)

    zeros = jnp.zeros((16,), jnp.float32)

    def zero_body(i, carry):
        deg_v[pl.ds(i * 16, 16)] = zeros
        return carry

    lax.fori_loop(0, N // 16, zero_body, 0)

    def edge_body(i, carry):
        idx = dst_v[pl.ds(i * 16, 16)]
        wv = w_v[pl.ds(i * 16, 16)]
        plsc.addupdate_scatter(deg_v, [idx], wv)
        return carry

    lax.fori_loop(0, EPW // 16, edge_body, 0)
    pltpu.sync_copy(deg_v, out_hbm.at[wid])


@functools.cache
def _make_deg():
    return functools.partial(
        pl.kernel,
        mesh=_mesh(),
        out_type=jax.ShapeDtypeStruct((NW, N), jnp.float32),
        scratch_types=[
            pltpu.VMEM((EPW,), jnp.int32),
            pltpu.VMEM((EPW,), jnp.float32),
            pltpu.VMEM((N,), jnp.float32),
        ],
        compiler_params=_SC_PARAMS,
    )(_deg_body)


@functools.cache
def _make_agg(F):
    nscale = F // 16

    @functools.partial(
        pl.kernel,
        mesh=_mesh(),
        out_type=jax.ShapeDtypeStruct((2, N, 128), jnp.float32),
        scratch_types=[
            pltpu.VMEM((SB, K), jnp.int32),
            pltpu.VMEM((SB, K), jnp.int32),
            pltpu.VMEM((SB, K), jnp.float32),
            pltpu.VMEM((K, 128), jnp.float32),
            pltpu.VMEM((K, 128), jnp.float32),
            pltpu.VMEM_SHARED((N, 128), jnp.float32),
            pltpu.SemaphoreType.DMA,
            pltpu.SemaphoreType.DMA,
        ],
        compiler_params=_SC_PARAMS,
    )
    def _agg(h_hbm, src_hbm, dst_hbm, w_hbm, zero_hbm, out_hbm,
             src_b, dst_b, w_b, rows0, rows1, acc_sh, sem0, sem1):
        c = lax.axis_index("c")
        s = lax.axis_index("s")
        wid = c * 16 + s
        row0 = pl.multiple_of(s * 632, 8)

        @pl.when(s < 15)
        def _zero_main():
            pltpu.sync_copy(zero_hbm.at[pl.ds(row0, 632)],
                            acc_sh.at[pl.ds(row0, 632)])

        @pl.when(s == 15)
        def _zero_tail():
            pltpu.sync_copy(zero_hbm.at[pl.ds(9480, 520)],
                            acc_sh.at[pl.ds(9480, 520)])

        plsc.subcore_barrier()

        def scale_and_scatter(jj, rows_v):
            idx_j = jnp.full((16,), jj, dtype=jnp.int32)

            def group(eg, carry):
                for i in range(EG):
                    idx_e = jnp.full((16,), i, dtype=jnp.int32) + eg * EG
                    wsp = plsc.load_gather(w_b, [idx_j, idx_e])
                    e0 = eg * EG + i
                    for k in range(nscale):
                        rows_v[e0, pl.ds(k * 16, 16)] = (
                            rows_v[e0, pl.ds(k * 16, 16)] * wsp)
                return carry

            lax.fori_loop(0, K // EG, group, 0)
            pltpu.sync_copy(rows_v, acc_sh.at[dst_b.at[jj]], add=True)

        def super_chunk(sb, carry):
            pltpu.sync_copy(src_hbm.at[wid, sb], src_b)
            pltpu.sync_copy(dst_hbm.at[wid, sb], dst_b)
            pltpu.sync_copy(w_hbm.at[wid, sb], w_b)
            cp = pltpu.async_copy(h_hbm.at[src_b.at[0]], rows0, sem0)

            def pair(jp, carry):
                jj0 = jp * 2
                pltpu.async_copy(h_hbm.at[src_b.at[jj0 + 1]], rows1, sem1)
                pltpu.make_async_copy(h_hbm.at[src_b.at[jj0]], rows0,
                                      sem0).wait()
                scale_and_scatter(jj0, rows0)
                pltpu.async_copy(h_hbm.at[src_b.at[jj0 + 2]], rows0, sem0)
                pltpu.make_async_copy(h_hbm.at[src_b.at[jj0 + 1]], rows1,
                                      sem1).wait()
                scale_and_scatter(jj0 + 1, rows1)
                return carry

            lax.fori_loop(0, (SB - 1) // 2, pair, carry)
            pltpu.make_async_copy(h_hbm.at[src_b.at[SB - 1]], rows0,
                                  sem0).wait()
            scale_and_scatter(SB - 1, rows0)
            return carry

        lax.fori_loop(0, NSB, super_chunk, 0)
        plsc.subcore_barrier()

        @pl.when(s < 15)
        def _out_main():
            pltpu.sync_copy(acc_sh.at[pl.ds(row0, 632)],
                            out_hbm.at[c, pl.ds(row0, 632)])

        @pl.when(s == 15)
        def _out_tail():
            pltpu.sync_copy(acc_sh.at[pl.ds(9480, 520)],
                            out_hbm.at[c, pl.ds(9480, 520)])

    return _agg


def _bn(v, g, b, eps=1e-5):
    mu = jnp.mean(v, axis=0)
    var = jnp.var(v, axis=0)
    return (v - mu) / jnp.sqrt(var + eps) * g + b


def _tc_call(body, out_shapes, *args):
    return pl.pallas_call(body, out_shape=out_shapes)(*args)


def _stage1_body(x_ref, degp_ref, g_ref, b_ref, w_ref, hs_ref, dis_ref):
    deg = jnp.sum(degp_ref[...], axis=0) + 1.0
    dis = lax.rsqrt(deg)
    a = _bn(x_ref[...], g_ref[...], b_ref[...])
    h = jnp.dot(a, w_ref[...], preferred_element_type=jnp.float32)
    hs_ref[...] = h * dis[:, None]
    dis_ref[...] = dis[:, None]


def _stage_mid_body(p0_ref, p1_ref, hs_ref, dis_ref, bias_ref, g_ref, b_ref,
                    w_ref, out_ref):
    dis = dis_ref[...]
    agg = dis * (p0_ref[...] + p1_ref[...] + hs_ref[...]) + bias_ref[...]
    a = jax.nn.relu(_bn(agg, g_ref[...], b_ref[...]))
    h = jnp.dot(a, w_ref[...], preferred_element_type=jnp.float32)
    out_ref[...] = h * dis


def _stage_final_body(p0_ref, p1_ref, hs_ref, dis_ref, bias_ref, g5_ref,
                      b5_ref, l1w_ref, l1b_ref, g6_ref, b6_ref, l2w_ref,
                      l2b_ref, out_ref):
    dis = dis_ref[...]
    agg = dis * (p0_ref[...] + p1_ref[...] + hs_ref[...]) + bias_ref[...]
    h = _bn(agg, g5_ref[...], b5_ref[...])
    h = jnp.dot(jax.nn.relu(h), l1w_ref[...],
                preferred_element_type=jnp.float32) + l1b_ref[...]
    h = _bn(h, g6_ref[...], b6_ref[...])
    out_ref[...] = jnp.dot(jax.nn.relu(h), l2w_ref[...],
                           preferred_element_type=jnp.float32) + l2b_ref[...]


def _padv(v):
    return jnp.pad(v, (0, 128 - v.shape[0]))


def _padm(m, cols=128):
    return jnp.pad(m, ((0, 128 - m.shape[0]), (0, cols - m.shape[1])))


def kernel(x, edge_weight, params, edge_index):
    p = params
    src4 = edge_index[0].reshape(NW, NSB, SB, K)
    dst4 = edge_index[1].reshape(NW, NSB, SB, K)
    w4 = edge_weight.reshape(NW, NSB, SB, K)
    dst2 = edge_index[1].reshape(NW, EPW)
    w2 = edge_weight.reshape(NW, EPW)
    zeros = jnp.zeros((N, 128), jnp.float32)

    deg_parts = _make_deg()(dst2, w2)

    hs1, dis = _tc_call(
        _stage1_body,
        (jax.ShapeDtypeStruct((N, 128), jnp.float32),
         jax.ShapeDtypeStruct((N, 1), jnp.float32)),
        x, deg_parts, p['bn1_g'], p['bn1_b'], p['W1'])

    def conv_step(hs, F, bias, g, b, w_next):
        parts = _make_agg(F)(hs, src4, dst4, w4, zeros)
        return _tc_call(
            _stage_mid_body,
            jax.ShapeDtypeStruct((N, 128), jnp.float32),
            parts[0], parts[1], hs, dis,
            _padv(bias), _padv(g), _padv(b), _padm(w_next))

    hs2 = conv_step(hs1, 128, p['b1'], p['bn2_g'], p['bn2_b'], p['W2'])
    hs3 = conv_step(hs2, 128, p['b2'], p['bn3_g'], p['bn3_b'], p['W3'])
    hs4 = conv_step(hs3, 64, p['b3'], p['bn4_g'], p['bn4_b'], p['W4'])

    parts4 = _make_agg(32)(hs4, src4, dst4, w4, zeros)
    out = _tc_call(
        _stage_final_body,
        jax.ShapeDtypeStruct((N, 40), jnp.float32),
        parts4[0], parts4[1], hs4, dis, _padv(p['b4']),
        _padv(p['bn5_g']), _padv(p['bn5_b']),
        _padm(p['lin1_W']), _padv(p['lin1_b']),
        _padv(p['bn6_g']), _padv(p['bn6_b']),
        _padm(p['lin2_W'], cols=40), p['lin2_b'])
    return out
